# baseline (speedup 1.0000x reference)
import jax
import jax.numpy as jnp
from jax.experimental import pallas as pl
from jax.experimental.pallas import tpu as pltpu

TOP_K = 8
NUM_EXPERTS = 64
HIDDEN_DIM = 4096

NBUF = 3
NPRIME = 2
CHUNK = 1024
NPIECE = 4
PIECE = CHUNK // NPIECE


def _router_body(x_hbm, w_ref, probs_ref, scores_ref, idx_ref, buf, sem, psem):
    n_tokens = x_hbm.shape[0]
    n_chunks = n_tokens // CHUNK
    last = n_chunks - 1

    def start(i):
        pltpu.make_async_copy(
            x_hbm.at[pl.ds(i * CHUNK, CHUNK), :], buf.at[i % NBUF], sem.at[i % NBUF]
        ).start()

    def wait(i):
        pltpu.make_async_copy(
            x_hbm.at[pl.ds(i * CHUNK, CHUNK), :], buf.at[i % NBUF], sem.at[i % NBUF]
        ).wait()

    def start_piece(q):
        pltpu.make_async_copy(
            x_hbm.at[pl.ds(last * CHUNK + q * PIECE, PIECE), :],
            buf.at[last % NBUF, pl.ds(q * PIECE, PIECE), :],
            psem.at[q],
        ).start()

    def wait_piece(q):
        pltpu.make_async_copy(
            x_hbm.at[pl.ds(last * CHUNK + q * PIECE, PIECE), :],
            buf.at[last % NBUF, pl.ds(q * PIECE, PIECE), :],
            psem.at[q],
        ).wait()

    def start_any(i):
        if i == last:
            for q in range(NPIECE):
                start_piece(q)
        else:
            start(i)

    iotas = {
        n: jax.lax.broadcasted_iota(jnp.int32, (NUM_EXPERTS, n), 0).astype(
            jnp.float32
        )
        for n in (CHUNK, PIECE)
    }

    def process(x, row0):
        nrows = x.shape[0]
        iota_t = iotas[nrows]
        logits_t = jax.lax.dot_general(
            w_ref[...], x, (((1,), (1,)), ((), ())),
            preferred_element_type=jnp.float32,
        )
        m = jnp.max(logits_t, axis=0, keepdims=True)
        e = jnp.exp(logits_t - m)
        s = jnp.sum(e, axis=0, keepdims=True)
        probs_t = e / s
        rows = pl.ds(row0, nrows)
        probs_ref[rows, :] = probs_t.T

        work = probs_t
        vals = []
        idxs = []
        for _ in range(TOP_K):
            mj = jnp.max(work, axis=0, keepdims=True)
            amj = jnp.min(
                jnp.where(work == mj, iota_t, float(NUM_EXPERTS)),
                axis=0,
                keepdims=True,
            )
            vals.append(mj)
            idxs.append(amj)
            work = jnp.where(iota_t == amj, -1.0, work)
        v_t = jnp.concatenate(vals, axis=0)
        scores_ref[:, rows] = v_t / jnp.sum(v_t, axis=0, keepdims=True)
        idx_ref[:, rows] = jnp.concatenate(idxs, axis=0).astype(jnp.int32)

    for i in range(NPRIME):
        start_any(i)

    for i in range(n_chunks):
        if i != last:
            wait(i)
            if i + NPRIME < n_chunks:
                start_any(i + NPRIME)
            process(buf[i % NBUF], i * CHUNK)
        else:
            for q in range(NPIECE):
                wait_piece(q)
                process(
                    buf[i % NBUF, pl.ds(q * PIECE, PIECE), :],
                    i * CHUNK + q * PIECE,
                )


def kernel(hidden_states, weight):
    x = hidden_states.reshape(-1, HIDDEN_DIM)
    n_tokens = x.shape[0]
    probs, scores_t, idx_t = pl.pallas_call(
        _router_body,
        in_specs=[
            pl.BlockSpec(memory_space=pl.ANY),
            pl.BlockSpec(memory_space=pltpu.MemorySpace.VMEM),
        ],
        out_specs=[
            pl.BlockSpec(memory_space=pltpu.MemorySpace.VMEM),
            pl.BlockSpec(memory_space=pltpu.MemorySpace.VMEM),
            pl.BlockSpec(memory_space=pltpu.MemorySpace.VMEM),
        ],
        out_shape=[
            jax.ShapeDtypeStruct((n_tokens, NUM_EXPERTS), jnp.float32),
            jax.ShapeDtypeStruct((TOP_K, n_tokens), jnp.float32),
            jax.ShapeDtypeStruct((TOP_K, n_tokens), jnp.int32),
        ],
        scratch_shapes=[
            pltpu.VMEM((NBUF, CHUNK, HIDDEN_DIM), jnp.float32),
            pltpu.SemaphoreType.DMA((NBUF,)),
            pltpu.SemaphoreType.DMA((NPIECE,)),
        ],
    )(x, weight)
    return probs, scores_t.T, idx_t.T


# speedup vs baseline: 1.0297x; 1.0297x over previous
import jax
import jax.numpy as jnp
from jax.experimental import pallas as pl
from jax.experimental.pallas import tpu as pltpu

TOP_K = 8
NUM_EXPERTS = 64
HIDDEN_DIM = 4096

NBUF = 3
NPRIME = 2
CHUNK = 1024


def _router_body(x_hbm, w_ref, probs_ref, scores_ref, idx_ref, buf, sem):
    n_tokens = x_hbm.shape[0]
    n_chunks = n_tokens // CHUNK

    def start(i):
        pltpu.make_async_copy(
            x_hbm.at[pl.ds(i * CHUNK, CHUNK), :], buf.at[i % NBUF], sem.at[i % NBUF]
        ).start()

    def wait(i):
        pltpu.make_async_copy(
            x_hbm.at[pl.ds(i * CHUNK, CHUNK), :], buf.at[i % NBUF], sem.at[i % NBUF]
        ).wait()

    iota_t = jax.lax.broadcasted_iota(jnp.int32, (NUM_EXPERTS, CHUNK), 0).astype(
        jnp.float32
    )

    for i in range(NPRIME):
        start(i)

    for i in range(n_chunks):
        wait(i)
        if i + NPRIME < n_chunks:
            start(i + NPRIME)
        x = buf[i % NBUF]
        logits_t = jax.lax.dot_general(
            w_ref[...], x, (((1,), (1,)), ((), ())),
            preferred_element_type=jnp.float32,
        )
        m = jnp.max(logits_t, axis=0, keepdims=True)
        e = jnp.exp(logits_t - m)
        s = jnp.sum(e, axis=0, keepdims=True)
        probs_t = e / s
        rows = pl.ds(i * CHUNK, CHUNK)
        probs_ref[rows, :] = probs_t.T

        work = probs_t
        vals = []
        idxs = []
        for _ in range(TOP_K):
            mj = jnp.max(work, axis=0, keepdims=True)
            amj = jnp.min(
                jnp.where(work == mj, iota_t, float(NUM_EXPERTS)),
                axis=0,
                keepdims=True,
            )
            vals.append(mj)
            idxs.append(amj)
            work = jnp.where(iota_t == amj, -1.0, work)
        v_t = jnp.concatenate(vals, axis=0)
        scores_ref[:, rows] = v_t / jnp.sum(v_t, axis=0, keepdims=True)
        idx_ref[:, rows] = jnp.concatenate(idxs, axis=0).astype(jnp.int32)


def kernel(hidden_states, weight):
    x = hidden_states.reshape(-1, HIDDEN_DIM)
    n_tokens = x.shape[0]
    probs, scores_t, idx_t = pl.pallas_call(
        _router_body,
        in_specs=[
            pl.BlockSpec(memory_space=pl.ANY),
            pl.BlockSpec(memory_space=pltpu.MemorySpace.VMEM),
        ],
        out_specs=[
            pl.BlockSpec(memory_space=pltpu.MemorySpace.VMEM),
            pl.BlockSpec(memory_space=pltpu.MemorySpace.VMEM),
            pl.BlockSpec(memory_space=pltpu.MemorySpace.VMEM),
        ],
        out_shape=[
            jax.ShapeDtypeStruct((n_tokens, NUM_EXPERTS), jnp.float32),
            jax.ShapeDtypeStruct((TOP_K, n_tokens), jnp.float32),
            jax.ShapeDtypeStruct((TOP_K, n_tokens), jnp.int32),
        ],
        scratch_shapes=[
            pltpu.VMEM((NBUF, CHUNK, HIDDEN_DIM), jnp.float32),
            pltpu.SemaphoreType.DMA((NBUF,)),
        ],
    )(x, weight)
    return probs, scores_t.T, idx_t.T
